# baseline (speedup 1.0000x reference)
import jax
import jax.numpy as jnp
from jax import lax
from jax.experimental import pallas as pl
from jax.experimental.pallas import tpu as pltpu
from jax.experimental.pallas import tpu_sc as plsc

_NUM_CORES = 2
_NUM_SUBCORES = 16
_NUM_WORKERS = _NUM_CORES * _NUM_SUBCORES
_ROWS_PER_CHUNK = 8
_NBUF = 2


def _gather_body(table_hbm, tok_hbm, out_hbm,
                 idx_bufs, row_bufs, idx_sems, gat_sems, out_sems):
    wid = lax.axis_index("s") * _NUM_CORES + lax.axis_index("c")
    n_tok_rows, seq = tok_hbm.shape
    rows_per_w = n_tok_rows // _NUM_WORKERS
    chunk = _ROWS_PER_CHUNK * seq
    row0 = wid * rows_per_w
    base = row0 * seq
    nchunks = rows_per_w // _ROWS_PER_CHUNK

    def idx_load(c, b):
        r0 = row0 + c * _ROWS_PER_CHUNK
        for j in range(_ROWS_PER_CHUNK):
            pltpu.async_copy(tok_hbm.at[r0 + j, :],
                             idx_bufs[b].at[pl.ds(j * seq, seq)],
                             idx_sems[b])

    def idx_wait(b):
        for j in range(_ROWS_PER_CHUNK):
            pltpu.make_async_copy(tok_hbm.at[0, :],
                                  idx_bufs[b].at[pl.ds(j * seq, seq)],
                                  idx_sems[b]).wait()

    def gather(b):
        pltpu.async_copy(table_hbm.at[idx_bufs[b]], row_bufs[b], gat_sems[b])

    def store(c, b):
        off = base + c * chunk
        pltpu.async_copy(row_bufs[b], out_hbm.at[pl.ds(off, chunk)],
                         out_sems[b])

    def steady(c, b, first_round):
        del first_round
        pltpu.make_async_copy(table_hbm.at[idx_bufs[b]], row_bufs[b],
                              gat_sems[b]).wait()

        @pl.when(c + _NBUF < nchunks)
        def _():
            idx_load(c + _NBUF, b)
        store(c, b)

        @pl.when(c + _NBUF < nchunks)
        def _():
            idx_wait(b)
            pltpu.make_async_copy(row_bufs[b], out_hbm.at[pl.ds(base, chunk)],
                                  out_sems[b]).wait()
            gather(b)

    for b in range(_NBUF):
        idx_load(b, b)
    for b in range(_NBUF):
        idx_wait(b)
        gather(b)
    for b in range(_NBUF):
        steady(b, b, first_round=True)

    def body(g, carry):
        for b in range(_NBUF):
            steady(_NBUF + g * _NBUF + b, b, first_round=False)
        return carry

    lax.fori_loop(0, (nchunks - _NBUF) // _NBUF, body, 0, unroll=False)

    for b in range(_NBUF):
        pltpu.make_async_copy(row_bufs[b], out_hbm.at[pl.ds(base, chunk)],
                              out_sems[b]).wait()


def kernel(token_ids, weight):
    n_rows, seq = token_ids.shape
    b = n_rows * seq
    d = weight.shape[1]
    if token_ids.dtype != jnp.int32:
        token_ids = token_ids.astype(jnp.int32)
    chunk = _ROWS_PER_CHUNK * seq
    mesh = plsc.VectorSubcoreMesh(core_axis_name="c", subcore_axis_name="s")
    gather = pl.kernel(
        _gather_body,
        mesh=mesh,
        out_type=jax.ShapeDtypeStruct((b, d), jnp.float32),
        scratch_types=[
            [pltpu.VMEM((chunk,), jnp.int32) for _ in range(_NBUF)],
            [pltpu.VMEM((chunk, d), jnp.float32) for _ in range(_NBUF)],
            [pltpu.SemaphoreType.DMA for _ in range(_NBUF)],
            [pltpu.SemaphoreType.DMA for _ in range(_NBUF)],
            [pltpu.SemaphoreType.DMA for _ in range(_NBUF)],
        ],
        compiler_params=pltpu.CompilerParams(use_tc_tiling_on_sc=False,
                                             needs_layout_passes=False),
    )
    out = gather(weight, token_ids)
    return out.reshape(n_rows, seq, d)
